# baseline (speedup 1.0000x reference)
import functools
import math

import jax
import jax.numpy as jnp
from jax import lax
from jax.experimental import pallas as pl
from jax.experimental.pallas import tpu as pltpu
from jax.experimental.pallas import tpu_sc as plsc

_CF, _RT = 0.25, 128
_LANES = 128
_NC, _NS, _L = 2, 16, 16
_NW = _NC * _NS


def _capacity_of(num_tokens):
    cap = math.ceil(_CF * num_tokens)
    cap = _RT * math.ceil(cap / _RT)
    return max(1, min(cap, num_tokens))


def _router_body(x_ref, wr_ref, dst0_ref, dst1_ref, src0_ref, src1_ref,
                 w0_ref, w1_ref, carry_ref, *, bm, cap, num_experts):
    @pl.when(pl.program_id(0) == 0)
    def _init():
        carry_ref[...] = jnp.zeros_like(carry_ref)

    x = x_ref[...]
    logits = jnp.dot(x, wr_ref[...], preferred_element_type=jnp.float32)
    col = lax.broadcasted_iota(jnp.int32, logits.shape, 1)
    valid = col < num_experts
    neg = jnp.float32(-jnp.inf)
    lm = jnp.where(valid, logits, neg)
    m1 = jnp.max(lm, axis=1, keepdims=True)
    i1 = jnp.min(jnp.where((lm == m1) & valid, col, _LANES), axis=1,
                 keepdims=True)
    oh0 = (col == i1).astype(jnp.float32)
    lm2 = jnp.where(col == i1, neg, lm)
    m2 = jnp.max(lm2, axis=1, keepdims=True)
    i2 = jnp.min(jnp.where((lm2 == m2) & valid, col, _LANES), axis=1,
                 keepdims=True)
    oh1 = (col == i2).astype(jnp.float32)
    ed = jnp.exp(m2 - m1)
    w0 = 1.0 / (1.0 + ed)
    w1 = ed / (1.0 + ed)
    r = lax.broadcasted_iota(jnp.int32, (bm, bm), 0)
    c = lax.broadcasted_iota(jnp.int32, (bm, bm), 1)
    tri = (r > c).astype(jnp.float32)
    oh = oh0 + oh1
    excl = jnp.dot(tri, oh, preferred_element_type=jnp.float32) + carry_ref[...]
    rank0 = jnp.sum(excl * oh0, axis=1).astype(jnp.int32)
    rank1 = jnp.sum(excl * oh1, axis=1).astype(jnp.int32)
    carry_ref[...] += jnp.sum(oh, axis=0, keepdims=True)

    e0 = i1[:, 0]
    e1 = i2[:, 0]
    k0 = rank0 < cap
    k1 = rank1 < cap
    f0 = e0 * cap + rank0
    f1 = e1 * cap + rank1
    trash = jnp.int32(num_experts * cap)
    dst0_ref[0, 0, :] = jnp.where(k0, f0, trash)
    dst1_ref[0, 0, :] = jnp.where(k1, f1, trash)
    src0_ref[0, 0, :] = jnp.where(k0, f0, 0)
    src1_ref[0, 0, :] = jnp.where(k1, f1, 0)
    w0_ref[0, 0, :] = jnp.where(k0, w0[:, 0], 0.0)
    w1_ref[0, 0, :] = jnp.where(k1, w1[:, 0], 0.0)


def _route(x2d, wr_pad, cap, num_experts, bm=512):
    n, h = x2d.shape
    nb = n // bm
    ispec = jnp.int32
    out_shapes = [jax.ShapeDtypeStruct((nb, 1, bm), ispec) for _ in range(4)]
    out_shapes += [jax.ShapeDtypeStruct((nb, 1, bm), jnp.float32) for _ in range(2)]
    small = pl.BlockSpec((1, 1, bm), lambda i: (i, 0, 0))
    return pl.pallas_call(
        functools.partial(_router_body, bm=bm, cap=cap, num_experts=num_experts),
        grid=(nb,),
        in_specs=[
            pl.BlockSpec((bm, h), lambda i: (i, 0)),
            pl.BlockSpec((h, _LANES), lambda i: (0, 0)),
        ],
        out_specs=[small] * 6,
        out_shape=out_shapes,
        scratch_shapes=[pltpu.VMEM((1, _LANES), jnp.float32)],
    )(x2d, wr_pad)


def _make_dispatch(n, h, rows_out, sub, nsub):
    mesh = plsc.VectorSubcoreMesh(core_axis_name="c", subcore_axis_name="s")

    @functools.partial(
        pl.kernel,
        out_type=jax.ShapeDtypeStruct((rows_out, h), jnp.float32),
        mesh=mesh,
        scratch_types=[
            pltpu.VMEM((sub, h), jnp.float32),
            pltpu.VMEM((sub,), jnp.int32),
            pltpu.VMEM((sub,), jnp.int32),
            pltpu.SemaphoreType.DMA,
        ],
    )
    def dispatch(x_hbm, dst0_hbm, dst1_hbm, einp_hbm, xbuf, d0v, d1v, sem):
        wid = lax.axis_index("s") * _NC + lax.axis_index("c")
        for s in range(nsub):
            base = wid * (sub * nsub) + s * sub
            pltpu.sync_copy(x_hbm.at[pl.ds(base, sub), :], xbuf)
            pltpu.sync_copy(dst0_hbm.at[wid, s], d0v)
            pltpu.sync_copy(dst1_hbm.at[wid, s], d1v)
            pltpu.async_copy(xbuf, einp_hbm.at[d0v], sem).wait()
            pltpu.async_copy(xbuf, einp_hbm.at[d1v], sem).wait()

    return dispatch


def _mlp_body(xe_ref, w1_ref, b1_ref, w2_ref, b2_ref, out_ref, *, nchunk=4):
    j = pl.program_id(1)
    x = xe_ref[...]
    bf = w1_ref.shape[2]
    cw = bf // nchunk
    contrib = None
    for c in range(nchunk):
        w1c = w1_ref[0, :, c * cw:(c + 1) * cw]
        hc = jnp.dot(x, w1c, preferred_element_type=jnp.float32)
        hc = jax.nn.gelu(hc + b1_ref[0, :, c * cw:(c + 1) * cw])
        pc = jnp.dot(hc, w2_ref[0, c * cw:(c + 1) * cw, :],
                     preferred_element_type=jnp.float32)
        contrib = pc if contrib is None else contrib + pc

    @pl.when(j == 0)
    def _first():
        out_ref[...] = contrib + b2_ref[0]

    @pl.when(j > 0)
    def _rest():
        out_ref[...] += contrib


def _mlp(einp, w1, b1, w2, b2, cap, bf=1024):
    num_experts, h, f = w1.shape
    nf = f // bf
    return pl.pallas_call(
        _mlp_body,
        grid=(num_experts, nf),
        in_specs=[
            pl.BlockSpec((cap, h), lambda e, j: (e, 0)),
            pl.BlockSpec((1, h, bf), lambda e, j: (e, 0, j)),
            pl.BlockSpec((1, 1, bf), lambda e, j: (e, 0, j)),
            pl.BlockSpec((1, bf, h), lambda e, j: (e, j, 0)),
            pl.BlockSpec((1, 1, h), lambda e, j: (e, 0, 0)),
        ],
        out_specs=pl.BlockSpec((cap, h), lambda e, j: (e, 0)),
        out_shape=jax.ShapeDtypeStruct((num_experts * cap, h), jnp.float32),
    )(einp, w1, b1.reshape(num_experts, 1, f), w2, b2.reshape(num_experts, 1, h))


def _make_combine(n, h, rows_in, sub, nsub):
    mesh = plsc.VectorSubcoreMesh(core_axis_name="c", subcore_axis_name="s")
    nvec = h // _L

    @functools.partial(
        pl.kernel,
        out_type=jax.ShapeDtypeStruct((n, h), jnp.float32),
        mesh=mesh,
        scratch_types=[
            pltpu.VMEM((sub, h), jnp.float32),
            pltpu.VMEM((sub, h), jnp.float32),
            pltpu.VMEM((sub, h), jnp.float32),
            pltpu.VMEM((sub,), jnp.int32),
            pltpu.VMEM((sub,), jnp.int32),
            pltpu.VMEM((sub, _L), jnp.float32),
            pltpu.VMEM((sub, _L), jnp.float32),
            pltpu.SemaphoreType.DMA,
        ],
    )
    def combine(eo_hbm, src0_hbm, src1_hbm, wt0_hbm, wt1_hbm, out_hbm,
                g0, g1, ob, i0v, i1v, w0v, w1v, sem):
        wid = lax.axis_index("s") * _NC + lax.axis_index("c")
        for s in range(nsub):
            base = wid * (sub * nsub) + s * sub
            pltpu.sync_copy(src0_hbm.at[wid, s], i0v)
            pltpu.sync_copy(src1_hbm.at[wid, s], i1v)
            pltpu.sync_copy(wt0_hbm.at[pl.ds(base, sub), :], w0v)
            pltpu.sync_copy(wt1_hbm.at[pl.ds(base, sub), :], w1v)
            pltpu.async_copy(eo_hbm.at[i0v], g0, sem).wait()
            pltpu.async_copy(eo_hbm.at[i1v], g1, sem).wait()

            def row_body(r, carry):
                w0vec = w0v[r, :]
                w1vec = w1v[r, :]
                zero = jnp.zeros((_L,), jnp.float32)
                unroll = 8

                def vec_body(v, c2):
                    for u in range(unroll):
                        off = (v * unroll + u) * _L
                        a = g0[r, pl.ds(off, _L)]
                        b = g1[r, pl.ds(off, _L)]
                        acc = (jnp.where(w0vec != 0.0, w0vec * a, zero)
                               + jnp.where(w1vec != 0.0, w1vec * b, zero))
                        ob[r, pl.ds(off, _L)] = acc
                    return c2

                lax.fori_loop(0, nvec // unroll, vec_body, 0)
                return carry

            lax.fori_loop(0, sub, row_body, 0)
            pltpu.sync_copy(ob, out_hbm.at[pl.ds(base, sub), :])

    return combine


def kernel(x, Wr, W1, b1, W2, b2):
    bsz, t, h = x.shape
    n = bsz * t
    num_experts = Wr.shape[1]
    cap = _capacity_of(n)
    x2d = x.reshape(n, h)
    wr_pad = jnp.zeros((h, _LANES), Wr.dtype).at[:, :num_experts].set(Wr)

    dst0, dst1, src0, src1, wt0, wt1 = _route(x2d, wr_pad, cap, num_experts)

    tpw = n // _NW
    sub_d = 64
    nsub_d = tpw // sub_d
    shp_d = (_NW, nsub_d, sub_d)
    einp = _make_dispatch(n, h, num_experts * cap + 8, sub_d, nsub_d)(
        x2d, dst0.reshape(shp_d), dst1.reshape(shp_d))

    eo = _mlp(einp, W1, b1, W2, b2, cap)

    sub_c = 32
    nsub_c = tpw // sub_c
    shp_c = (_NW, nsub_c, sub_c)
    wt0x = jnp.broadcast_to(wt0.reshape(n, 1), (n, _L))
    wt1x = jnp.broadcast_to(wt1.reshape(n, 1), (n, _L))
    out = _make_combine(n, h, num_experts * cap, sub_c, nsub_c)(
        eo, src0.reshape(shp_c), src1.reshape(shp_c), wt0x, wt1x)
    return out.reshape(bsz, t, h)


# speedup vs baseline: 1.3520x; 1.3520x over previous
import functools
import math

import jax
import jax.numpy as jnp
from jax import lax
from jax.experimental import pallas as pl
from jax.experimental.pallas import tpu as pltpu
from jax.experimental.pallas import tpu_sc as plsc

_CF, _RT = 0.25, 128
_LANES = 128
_NC, _NS, _L = 2, 16, 16
_NW = _NC * _NS


def _capacity_of(num_tokens):
    cap = math.ceil(_CF * num_tokens)
    cap = _RT * math.ceil(cap / _RT)
    return max(1, min(cap, num_tokens))


def _router_body(x_ref, wr_ref, dst0_ref, dst1_ref, src0_ref, src1_ref,
                 w0_ref, w1_ref, carry_ref, *, bm, cap, num_experts):
    @pl.when(pl.program_id(0) == 0)
    def _init():
        carry_ref[...] = jnp.zeros_like(carry_ref)

    x = x_ref[...]
    logits = jnp.dot(x, wr_ref[...], preferred_element_type=jnp.float32)
    col = lax.broadcasted_iota(jnp.int32, logits.shape, 1)
    valid = col < num_experts
    neg = jnp.float32(-jnp.inf)
    lm = jnp.where(valid, logits, neg)
    m1 = jnp.max(lm, axis=1, keepdims=True)
    i1 = jnp.min(jnp.where((lm == m1) & valid, col, _LANES), axis=1,
                 keepdims=True)
    oh0 = (col == i1).astype(jnp.float32)
    lm2 = jnp.where(col == i1, neg, lm)
    m2 = jnp.max(lm2, axis=1, keepdims=True)
    i2 = jnp.min(jnp.where((lm2 == m2) & valid, col, _LANES), axis=1,
                 keepdims=True)
    oh1 = (col == i2).astype(jnp.float32)
    ed = jnp.exp(m2 - m1)
    w0 = 1.0 / (1.0 + ed)
    w1 = ed / (1.0 + ed)
    r = lax.broadcasted_iota(jnp.int32, (bm, bm), 0)
    c = lax.broadcasted_iota(jnp.int32, (bm, bm), 1)
    tri = (r > c).astype(jnp.float32)
    oh = oh0 + oh1
    excl = jnp.dot(tri, oh, preferred_element_type=jnp.float32) + carry_ref[...]
    rank0 = jnp.sum(excl * oh0, axis=1).astype(jnp.int32)
    rank1 = jnp.sum(excl * oh1, axis=1).astype(jnp.int32)
    carry_ref[...] += jnp.sum(oh, axis=0, keepdims=True)

    e0 = i1[:, 0]
    e1 = i2[:, 0]
    k0 = rank0 < cap
    k1 = rank1 < cap
    f0 = e0 * cap + rank0
    f1 = e1 * cap + rank1
    trash = jnp.int32(num_experts * cap)
    dst0_ref[0, 0, :] = jnp.where(k0, f0, trash)
    dst1_ref[0, 0, :] = jnp.where(k1, f1, trash)
    src0_ref[0, 0, :] = jnp.where(k0, f0, 0)
    src1_ref[0, 0, :] = jnp.where(k1, f1, 0)
    w0_ref[0, 0, :] = jnp.where(k0, w0[:, 0], 0.0)
    w1_ref[0, 0, :] = jnp.where(k1, w1[:, 0], 0.0)


def _route(x2d, wr_pad, cap, num_experts, bm=512):
    n, h = x2d.shape
    nb = n // bm
    ispec = jnp.int32
    out_shapes = [jax.ShapeDtypeStruct((nb, 1, bm), ispec) for _ in range(4)]
    out_shapes += [jax.ShapeDtypeStruct((nb, 1, bm), jnp.float32) for _ in range(2)]
    small = pl.BlockSpec((1, 1, bm), lambda i: (i, 0, 0))
    return pl.pallas_call(
        functools.partial(_router_body, bm=bm, cap=cap, num_experts=num_experts),
        grid=(nb,),
        in_specs=[
            pl.BlockSpec((bm, h), lambda i: (i, 0)),
            pl.BlockSpec((h, _LANES), lambda i: (0, 0)),
        ],
        out_specs=[small] * 6,
        out_shape=out_shapes,
        scratch_shapes=[pltpu.VMEM((1, _LANES), jnp.float32)],
    )(x2d, wr_pad)


def _make_dispatch(n, h, rows_out, sub, nsub):
    mesh = plsc.VectorSubcoreMesh(core_axis_name="c", subcore_axis_name="s")

    @functools.partial(
        pl.kernel,
        out_type=jax.ShapeDtypeStruct((rows_out, h), jnp.float32),
        mesh=mesh,
        scratch_types=[
            pltpu.VMEM((sub, h), jnp.float32),
            pltpu.VMEM((sub,), jnp.int32),
            pltpu.VMEM((sub,), jnp.int32),
            pltpu.SemaphoreType.DMA,
        ],
    )
    def dispatch(x_hbm, dst0_hbm, dst1_hbm, einp_hbm, xbuf, d0v, d1v, sem):
        wid = lax.axis_index("s") * _NC + lax.axis_index("c")
        for s in range(nsub):
            base = wid * (sub * nsub) + s * sub
            pltpu.sync_copy(x_hbm.at[pl.ds(base, sub), :], xbuf)
            pltpu.sync_copy(dst0_hbm.at[wid, s], d0v)
            pltpu.sync_copy(dst1_hbm.at[wid, s], d1v)
            pltpu.async_copy(xbuf, einp_hbm.at[d0v], sem).wait()
            pltpu.async_copy(xbuf, einp_hbm.at[d1v], sem).wait()

    return dispatch


def _mlp_body(xe_ref, w1_ref, b1_ref, w2_ref, b2_ref, out_ref):
    j = pl.program_id(1)
    x = xe_ref[...]
    h = jnp.dot(x, w1_ref[0], preferred_element_type=jnp.float32) + b1_ref[0]
    h = jax.nn.gelu(h)
    contrib = jnp.dot(h, w2_ref[0], preferred_element_type=jnp.float32)

    @pl.when(j == 0)
    def _first():
        out_ref[...] = contrib + b2_ref[0]

    @pl.when(j > 0)
    def _rest():
        out_ref[...] += contrib


def _mlp(einp, w1, b1, w2, b2, cap, bf=1024):
    num_experts, h, f = w1.shape
    nf = f // bf
    return pl.pallas_call(
        _mlp_body,
        grid=(num_experts, nf),
        in_specs=[
            pl.BlockSpec((cap, h), lambda e, j: (e, 0)),
            pl.BlockSpec((1, h, bf), lambda e, j: (e, 0, j)),
            pl.BlockSpec((1, 1, bf), lambda e, j: (e, 0, j)),
            pl.BlockSpec((1, bf, h), lambda e, j: (e, j, 0)),
            pl.BlockSpec((1, 1, h), lambda e, j: (e, 0, 0)),
        ],
        out_specs=pl.BlockSpec((cap, h), lambda e, j: (e, 0)),
        out_shape=jax.ShapeDtypeStruct((num_experts * cap, h), jnp.float32),
    )(einp, w1, b1.reshape(num_experts, 1, f), w2, b2.reshape(num_experts, 1, h))


def _make_combine(n, h, rows_in, sub, nsub):
    mesh = plsc.VectorSubcoreMesh(core_axis_name="c", subcore_axis_name="s")
    nvec = h // _L

    @functools.partial(
        pl.kernel,
        out_type=jax.ShapeDtypeStruct((n, h), jnp.float32),
        mesh=mesh,
        scratch_types=[
            pltpu.VMEM((sub, h), jnp.float32),
            pltpu.VMEM((sub, h), jnp.float32),
            pltpu.VMEM((sub, h), jnp.float32),
            pltpu.VMEM((sub, h), jnp.float32),
            pltpu.VMEM((sub, h), jnp.float32),
            pltpu.VMEM((sub, h), jnp.float32),
            pltpu.VMEM((sub,), jnp.int32),
            pltpu.VMEM((sub,), jnp.int32),
            pltpu.VMEM((sub,), jnp.int32),
            pltpu.VMEM((sub,), jnp.int32),
            pltpu.VMEM((sub, 2 * _L), jnp.float32),
            pltpu.VMEM((sub, 2 * _L), jnp.float32),
            pltpu.SemaphoreType.DMA,
            pltpu.SemaphoreType.DMA,
            pltpu.SemaphoreType.DMA,
            pltpu.SemaphoreType.DMA,
            pltpu.SemaphoreType.DMA,
        ],
    )
    def combine(eo_hbm, src0_hbm, src1_hbm, w_hbm, out_hbm,
                g0a, g0b, g1a, g1b, oba, obb, i0a, i0b, i1a, i1b, wva, wvb,
                sga, sgb, ssa, ssb, si):
        wid = lax.axis_index("s") * _NC + lax.axis_index("c")
        g0 = [g0a, g0b]
        g1 = [g1a, g1b]
        ob = [oba, obb]
        i0 = [i0a, i0b]
        i1 = [i1a, i1b]
        wv = [wva, wvb]
        sg = [sga, sgb]
        ss = [ssa, ssb]

        def load_idx(s, p):
            h1 = pltpu.async_copy(src0_hbm.at[wid, s], i0[p], si)
            h2 = pltpu.async_copy(src1_hbm.at[wid, s], i1[p], si)
            h3 = pltpu.async_copy(
                w_hbm.at[pl.ds(wid * (sub * nsub) + s * sub, sub), :], wv[p], si)
            h1.wait()
            h2.wait()
            h3.wait()

        def fire_gathers(p):
            c0 = pltpu.async_copy(eo_hbm.at[i0[p]], g0[p], sg[p])
            c1 = pltpu.async_copy(eo_hbm.at[i1[p]], g1[p], sg[p])
            return c0, c1

        def compute(p):
            def row_body(r, carry):
                w0vec = wv[p][r, 0:_L]
                w1vec = wv[p][r, _L:2 * _L]
                zero = jnp.zeros((_L,), jnp.float32)

                def vec_body(v, c2):
                    a = g0[p][r, pl.ds(v * _L, _L)]
                    b = g1[p][r, pl.ds(v * _L, _L)]
                    acc = (jnp.where(w0vec != 0.0, w0vec * a, zero)
                           + jnp.where(w1vec != 0.0, w1vec * b, zero))
                    ob[p][r, pl.ds(v * _L, _L)] = acc
                    return c2

                lax.fori_loop(0, nvec, vec_body, 0)
                return carry

            lax.fori_loop(0, sub, row_body, 0)

        load_idx(0, 0)
        pend = {0: fire_gathers(0)}
        store_pend = [None, None]
        for s in range(nsub):
            p = s % 2
            if s + 1 < nsub:
                load_idx(s + 1, 1 - p)
                pend[s + 1] = fire_gathers(1 - p)
            c0, c1 = pend.pop(s)
            c0.wait()
            c1.wait()
            if store_pend[p] is not None:
                store_pend[p].wait()
                store_pend[p] = None
            compute(p)
            base = wid * (sub * nsub) + s * sub
            store_pend[p] = pltpu.async_copy(
                ob[p], out_hbm.at[pl.ds(base, sub), :], ss[p])
        for sp in store_pend:
            if sp is not None:
                sp.wait()

    return combine


def kernel(x, Wr, W1, b1, W2, b2):
    bsz, t, h = x.shape
    n = bsz * t
    num_experts = Wr.shape[1]
    cap = _capacity_of(n)
    x2d = x.reshape(n, h)
    wr_pad = jnp.zeros((h, _LANES), Wr.dtype).at[:, :num_experts].set(Wr)

    dst0, dst1, src0, src1, wt0, wt1 = _route(x2d, wr_pad, cap, num_experts)

    tpw = n // _NW
    sub_d = 64
    nsub_d = tpw // sub_d
    shp_d = (_NW, nsub_d, sub_d)
    einp = _make_dispatch(n, h, num_experts * cap + 8, sub_d, nsub_d)(
        x2d, dst0.reshape(shp_d), dst1.reshape(shp_d))

    eo = _mlp(einp, W1, b1, W2, b2, cap)

    sub_c = 16
    nsub_c = tpw // sub_c
    shp_c = (_NW, nsub_c, sub_c)
    wpack = jnp.concatenate(
        [jnp.broadcast_to(wt0.reshape(n, 1), (n, _L)),
         jnp.broadcast_to(wt1.reshape(n, 1), (n, _L))], axis=1)
    out = _make_combine(n, h, num_experts * cap, sub_c, nsub_c)(
        eo, src0.reshape(shp_c), src1.reshape(shp_c), wpack)
    return out.reshape(bsz, t, h)


# speedup vs baseline: 1.3568x; 1.0035x over previous
import functools
import math

import jax
import jax.numpy as jnp
from jax import lax
from jax.experimental import pallas as pl
from jax.experimental.pallas import tpu as pltpu
from jax.experimental.pallas import tpu_sc as plsc

_CF, _RT = 0.25, 128
_LANES = 128
_NC, _NS, _L = 2, 16, 16
_NW = _NC * _NS


def _capacity_of(num_tokens):
    cap = math.ceil(_CF * num_tokens)
    cap = _RT * math.ceil(cap / _RT)
    return max(1, min(cap, num_tokens))


def _router_body(x_ref, wr_ref, dst0_ref, dst1_ref, src0_ref, src1_ref,
                 w0_ref, w1_ref, carry_ref, *, bm, cap, num_experts):
    @pl.when(pl.program_id(0) == 0)
    def _init():
        carry_ref[...] = jnp.zeros_like(carry_ref)

    x = x_ref[...]
    logits = jnp.dot(x, wr_ref[...], preferred_element_type=jnp.float32)
    col = lax.broadcasted_iota(jnp.int32, logits.shape, 1)
    valid = col < num_experts
    neg = jnp.float32(-jnp.inf)
    lm = jnp.where(valid, logits, neg)
    m1 = jnp.max(lm, axis=1, keepdims=True)
    i1 = jnp.min(jnp.where((lm == m1) & valid, col, _LANES), axis=1,
                 keepdims=True)
    oh0 = (col == i1).astype(jnp.float32)
    lm2 = jnp.where(col == i1, neg, lm)
    m2 = jnp.max(lm2, axis=1, keepdims=True)
    i2 = jnp.min(jnp.where((lm2 == m2) & valid, col, _LANES), axis=1,
                 keepdims=True)
    oh1 = (col == i2).astype(jnp.float32)
    ed = jnp.exp(m2 - m1)
    w0 = 1.0 / (1.0 + ed)
    w1 = ed / (1.0 + ed)
    r = lax.broadcasted_iota(jnp.int32, (bm, bm), 0)
    c = lax.broadcasted_iota(jnp.int32, (bm, bm), 1)
    tri = (r > c).astype(jnp.float32)
    oh = oh0 + oh1
    excl = jnp.dot(tri, oh, preferred_element_type=jnp.float32) + carry_ref[...]
    rank0 = jnp.sum(excl * oh0, axis=1).astype(jnp.int32)
    rank1 = jnp.sum(excl * oh1, axis=1).astype(jnp.int32)
    carry_ref[...] += jnp.sum(oh, axis=0, keepdims=True)

    e0 = i1[:, 0]
    e1 = i2[:, 0]
    k0 = rank0 < cap
    k1 = rank1 < cap
    f0 = e0 * cap + rank0
    f1 = e1 * cap + rank1
    trash = jnp.int32(num_experts * cap)
    dst0_ref[0, 0, :] = jnp.where(k0, f0, trash)
    dst1_ref[0, 0, :] = jnp.where(k1, f1, trash)
    src0_ref[0, 0, :] = jnp.where(k0, f0, 0)
    src1_ref[0, 0, :] = jnp.where(k1, f1, 0)
    w0_ref[0, 0, :] = jnp.where(k0, w0[:, 0], 0.0)
    w1_ref[0, 0, :] = jnp.where(k1, w1[:, 0], 0.0)


def _route(x2d, wr_pad, cap, num_experts, bm=512):
    n, h = x2d.shape
    nb = n // bm
    ispec = jnp.int32
    out_shapes = [jax.ShapeDtypeStruct((nb, 1, bm), ispec) for _ in range(4)]
    out_shapes += [jax.ShapeDtypeStruct((nb, 1, bm), jnp.float32) for _ in range(2)]
    small = pl.BlockSpec((1, 1, bm), lambda i: (i, 0, 0))
    return pl.pallas_call(
        functools.partial(_router_body, bm=bm, cap=cap, num_experts=num_experts),
        grid=(nb,),
        in_specs=[
            pl.BlockSpec((bm, h), lambda i: (i, 0)),
            pl.BlockSpec((h, _LANES), lambda i: (0, 0)),
        ],
        out_specs=[small] * 6,
        out_shape=out_shapes,
        scratch_shapes=[pltpu.VMEM((1, _LANES), jnp.float32)],
    )(x2d, wr_pad)


def _make_dispatch(n, h, rows_out, sub, nsub):
    mesh = plsc.VectorSubcoreMesh(core_axis_name="c", subcore_axis_name="s")

    @functools.partial(
        pl.kernel,
        out_type=jax.ShapeDtypeStruct((rows_out, h), jnp.float32),
        mesh=mesh,
        scratch_types=[
            pltpu.VMEM((sub, h), jnp.float32),
            pltpu.VMEM((sub, h), jnp.float32),
            pltpu.VMEM((sub,), jnp.int32),
            pltpu.VMEM((sub,), jnp.int32),
            pltpu.VMEM((sub,), jnp.int32),
            pltpu.VMEM((sub,), jnp.int32),
            pltpu.SemaphoreType.DMA,
            pltpu.SemaphoreType.DMA,
            pltpu.SemaphoreType.DMA,
            pltpu.SemaphoreType.DMA,
            pltpu.SemaphoreType.DMA,
        ],
    )
    def dispatch(x_hbm, dst0_hbm, dst1_hbm, einp_hbm,
                 xba, xbb, d0a, d0b, d1a, d1b, sxa, sxb, sca, scb, si):
        wid = lax.axis_index("s") * _NC + lax.axis_index("c")
        xb = [xba, xbb]
        d0 = [d0a, d0b]
        d1 = [d1a, d1b]
        sx = [sxa, sxb]
        sc = [sca, scb]

        def fire_load(s, p):
            base = wid * (sub * nsub) + s * sub
            hx = pltpu.async_copy(x_hbm.at[pl.ds(base, sub), :], xb[p], sx[p])
            h0 = pltpu.async_copy(dst0_hbm.at[wid, s], d0[p], si)
            h1 = pltpu.async_copy(dst1_hbm.at[wid, s], d1[p], si)
            return hx, h0, h1

        pend = {0: fire_load(0, 0)}
        scat_pend = [None, None]
        for s in range(nsub):
            p = s % 2
            if s + 1 < nsub:
                if scat_pend[1 - p] is not None:
                    a, b = scat_pend[1 - p]
                    a.wait()
                    b.wait()
                    scat_pend[1 - p] = None
                pend[s + 1] = fire_load(s + 1, 1 - p)
            hx, h0, h1 = pend.pop(s)
            hx.wait()
            h0.wait()
            h1.wait()
            sc0 = pltpu.async_copy(xb[p], einp_hbm.at[d0[p]], sc[p])
            sc1 = pltpu.async_copy(xb[p], einp_hbm.at[d1[p]], sc[p])
            scat_pend[p] = (sc0, sc1)
        for sp in scat_pend:
            if sp is not None:
                sp[0].wait()
                sp[1].wait()

    return dispatch


def _mlp_body(xe_ref, w1_ref, b1_ref, w2_ref, b2_ref, out_ref):
    j = pl.program_id(1)
    x = xe_ref[...]
    h = jnp.dot(x, w1_ref[0], preferred_element_type=jnp.float32) + b1_ref[0]
    h = jax.nn.gelu(h)
    contrib = jnp.dot(h, w2_ref[0], preferred_element_type=jnp.float32)

    @pl.when(j == 0)
    def _first():
        out_ref[...] = contrib + b2_ref[0]

    @pl.when(j > 0)
    def _rest():
        out_ref[...] += contrib


def _mlp(einp, w1, b1, w2, b2, cap, bf=1024):
    num_experts, h, f = w1.shape
    nf = f // bf
    return pl.pallas_call(
        _mlp_body,
        grid=(num_experts, nf),
        in_specs=[
            pl.BlockSpec((cap, h), lambda e, j: (e, 0)),
            pl.BlockSpec((1, h, bf), lambda e, j: (e, 0, j)),
            pl.BlockSpec((1, 1, bf), lambda e, j: (e, 0, j)),
            pl.BlockSpec((1, bf, h), lambda e, j: (e, j, 0)),
            pl.BlockSpec((1, 1, h), lambda e, j: (e, 0, 0)),
        ],
        out_specs=pl.BlockSpec((cap, h), lambda e, j: (e, 0)),
        out_shape=jax.ShapeDtypeStruct((num_experts * cap, h), jnp.float32),
    )(einp, w1, b1.reshape(num_experts, 1, f), w2, b2.reshape(num_experts, 1, h))


def _make_combine(n, h, rows_in, sub, nsub):
    mesh = plsc.VectorSubcoreMesh(core_axis_name="c", subcore_axis_name="s")
    nvec = h // _L

    @functools.partial(
        pl.kernel,
        out_type=jax.ShapeDtypeStruct((n, h), jnp.float32),
        mesh=mesh,
        scratch_types=[
            pltpu.VMEM((sub, h), jnp.float32),
            pltpu.VMEM((sub, h), jnp.float32),
            pltpu.VMEM((sub, h), jnp.float32),
            pltpu.VMEM((sub, h), jnp.float32),
            pltpu.VMEM((sub, h), jnp.float32),
            pltpu.VMEM((sub, h), jnp.float32),
            pltpu.VMEM((sub,), jnp.int32),
            pltpu.VMEM((sub,), jnp.int32),
            pltpu.VMEM((sub,), jnp.int32),
            pltpu.VMEM((sub,), jnp.int32),
            pltpu.VMEM((sub, 2 * _L), jnp.float32),
            pltpu.VMEM((sub, 2 * _L), jnp.float32),
            pltpu.SemaphoreType.DMA,
            pltpu.SemaphoreType.DMA,
            pltpu.SemaphoreType.DMA,
            pltpu.SemaphoreType.DMA,
            pltpu.SemaphoreType.DMA,
        ],
    )
    def combine(eo_hbm, src0_hbm, src1_hbm, w_hbm, out_hbm,
                g0a, g0b, g1a, g1b, oba, obb, i0a, i0b, i1a, i1b, wva, wvb,
                sga, sgb, ssa, ssb, si):
        wid = lax.axis_index("s") * _NC + lax.axis_index("c")
        g0 = [g0a, g0b]
        g1 = [g1a, g1b]
        ob = [oba, obb]
        i0 = [i0a, i0b]
        i1 = [i1a, i1b]
        wv = [wva, wvb]
        sg = [sga, sgb]
        ss = [ssa, ssb]

        def load_idx(s, p):
            h1 = pltpu.async_copy(src0_hbm.at[wid, s], i0[p], si)
            h2 = pltpu.async_copy(src1_hbm.at[wid, s], i1[p], si)
            h3 = pltpu.async_copy(
                w_hbm.at[pl.ds(wid * (sub * nsub) + s * sub, sub), :], wv[p], si)
            h1.wait()
            h2.wait()
            h3.wait()

        def fire_gathers(p):
            c0 = pltpu.async_copy(eo_hbm.at[i0[p]], g0[p], sg[p])
            c1 = pltpu.async_copy(eo_hbm.at[i1[p]], g1[p], sg[p])
            return c0, c1

        def compute(p):
            def row_body(r, carry):
                w0vec = wv[p][r, 0:_L]
                w1vec = wv[p][r, _L:2 * _L]
                zero = jnp.zeros((_L,), jnp.float32)

                def vec_body(v, c2):
                    a = g0[p][r, pl.ds(v * _L, _L)]
                    b = g1[p][r, pl.ds(v * _L, _L)]
                    acc = (jnp.where(w0vec != 0.0, w0vec * a, zero)
                           + jnp.where(w1vec != 0.0, w1vec * b, zero))
                    ob[p][r, pl.ds(v * _L, _L)] = acc
                    return c2

                lax.fori_loop(0, nvec, vec_body, 0)
                return carry

            lax.fori_loop(0, sub, row_body, 0)

        load_idx(0, 0)
        pend = {0: fire_gathers(0)}
        store_pend = [None, None]
        for s in range(nsub):
            p = s % 2
            if s + 1 < nsub:
                load_idx(s + 1, 1 - p)
                pend[s + 1] = fire_gathers(1 - p)
            c0, c1 = pend.pop(s)
            c0.wait()
            c1.wait()
            if store_pend[p] is not None:
                store_pend[p].wait()
                store_pend[p] = None
            compute(p)
            base = wid * (sub * nsub) + s * sub
            store_pend[p] = pltpu.async_copy(
                ob[p], out_hbm.at[pl.ds(base, sub), :], ss[p])
        for sp in store_pend:
            if sp is not None:
                sp.wait()

    return combine


def kernel(x, Wr, W1, b1, W2, b2):
    bsz, t, h = x.shape
    n = bsz * t
    num_experts = Wr.shape[1]
    cap = _capacity_of(n)
    x2d = x.reshape(n, h)
    wr_pad = jnp.zeros((h, _LANES), Wr.dtype).at[:, :num_experts].set(Wr)

    dst0, dst1, src0, src1, wt0, wt1 = _route(x2d, wr_pad, cap, num_experts)

    tpw = n // _NW
    sub_d = 32
    nsub_d = tpw // sub_d
    shp_d = (_NW, nsub_d, sub_d)
    einp = _make_dispatch(n, h, num_experts * cap + 8, sub_d, nsub_d)(
        x2d, dst0.reshape(shp_d), dst1.reshape(shp_d))

    eo = _mlp(einp, W1, b1, W2, b2, cap)

    sub_c = 16
    nsub_c = tpw // sub_c
    shp_c = (_NW, nsub_c, sub_c)
    wpack = jnp.concatenate(
        [jnp.broadcast_to(wt0.reshape(n, 1), (n, _L)),
         jnp.broadcast_to(wt1.reshape(n, 1), (n, _L))], axis=1)
    out = _make_combine(n, h, num_experts * cap, sub_c, nsub_c)(
        eo, src0.reshape(shp_c), src1.reshape(shp_c), wpack)
    return out.reshape(bsz, t, h)


# speedup vs baseline: 1.3791x; 1.0165x over previous
import functools
import math

import jax
import jax.numpy as jnp
from jax import lax
from jax.experimental import pallas as pl
from jax.experimental.pallas import tpu as pltpu
from jax.experimental.pallas import tpu_sc as plsc

_CF, _RT = 0.25, 128
_LANES = 128
_NC, _NS, _L = 2, 16, 16
_NW = _NC * _NS


def _capacity_of(num_tokens):
    cap = math.ceil(_CF * num_tokens)
    cap = _RT * math.ceil(cap / _RT)
    return max(1, min(cap, num_tokens))


def _router_body(x_ref, wr_ref, dst0_ref, dst1_ref, src0_ref, src1_ref,
                 w0_ref, w1_ref, carry_ref, *, bm, cap, num_experts):
    @pl.when(pl.program_id(0) == 0)
    def _init():
        carry_ref[...] = jnp.zeros_like(carry_ref)

    x = x_ref[...]
    logits = jnp.dot(x, wr_ref[...], preferred_element_type=jnp.float32)
    col = lax.broadcasted_iota(jnp.int32, logits.shape, 1)
    valid = col < num_experts
    neg = jnp.float32(-jnp.inf)
    lm = jnp.where(valid, logits, neg)
    m1 = jnp.max(lm, axis=1, keepdims=True)
    i1 = jnp.min(jnp.where((lm == m1) & valid, col, _LANES), axis=1,
                 keepdims=True)
    oh0 = (col == i1).astype(jnp.float32)
    lm2 = jnp.where(col == i1, neg, lm)
    m2 = jnp.max(lm2, axis=1, keepdims=True)
    i2 = jnp.min(jnp.where((lm2 == m2) & valid, col, _LANES), axis=1,
                 keepdims=True)
    oh1 = (col == i2).astype(jnp.float32)
    ed = jnp.exp(m2 - m1)
    w0 = 1.0 / (1.0 + ed)
    w1 = ed / (1.0 + ed)
    r = lax.broadcasted_iota(jnp.int32, (bm, bm), 0)
    c = lax.broadcasted_iota(jnp.int32, (bm, bm), 1)
    tri = (r > c).astype(jnp.float32)
    oh = oh0 + oh1
    excl = jnp.dot(tri, oh, preferred_element_type=jnp.float32) + carry_ref[...]
    rank0 = jnp.sum(excl * oh0, axis=1).astype(jnp.int32)
    rank1 = jnp.sum(excl * oh1, axis=1).astype(jnp.int32)
    carry_ref[...] += jnp.sum(oh, axis=0, keepdims=True)

    e0 = i1[:, 0]
    e1 = i2[:, 0]
    k0 = rank0 < cap
    k1 = rank1 < cap
    f0 = e0 * cap + rank0
    f1 = e1 * cap + rank1
    trash = jnp.int32(num_experts * cap)
    dst0_ref[0, 0, :] = jnp.where(k0, f0, trash)
    dst1_ref[0, 0, :] = jnp.where(k1, f1, trash)
    src0_ref[0, 0, :] = jnp.where(k0, f0, 0)
    src1_ref[0, 0, :] = jnp.where(k1, f1, 0)
    w0_ref[0, 0, :] = jnp.where(k0, w0[:, 0], 0.0)
    w1_ref[0, 0, :] = jnp.where(k1, w1[:, 0], 0.0)


def _route(x2d, wr_pad, cap, num_experts, bm=512):
    n, h = x2d.shape
    nb = n // bm
    ispec = jnp.int32
    out_shapes = [jax.ShapeDtypeStruct((nb, 1, bm), ispec) for _ in range(4)]
    out_shapes += [jax.ShapeDtypeStruct((nb, 1, bm), jnp.float32) for _ in range(2)]
    small = pl.BlockSpec((1, 1, bm), lambda i: (i, 0, 0))
    return pl.pallas_call(
        functools.partial(_router_body, bm=bm, cap=cap, num_experts=num_experts),
        grid=(nb,),
        in_specs=[
            pl.BlockSpec((bm, h), lambda i: (i, 0)),
            pl.BlockSpec((h, _LANES), lambda i: (0, 0)),
        ],
        out_specs=[small] * 6,
        out_shape=out_shapes,
        scratch_shapes=[pltpu.VMEM((1, _LANES), jnp.float32)],
    )(x2d, wr_pad)


def _make_dispatch(n, h, rows_out, sub, nsub):
    mesh = plsc.VectorSubcoreMesh(core_axis_name="c", subcore_axis_name="s")

    @functools.partial(
        pl.kernel,
        out_type=jax.ShapeDtypeStruct((rows_out, h), jnp.float32),
        mesh=mesh,
        scratch_types=[
            pltpu.VMEM((sub, h), jnp.float32),
            pltpu.VMEM((sub, h), jnp.float32),
            pltpu.VMEM((sub,), jnp.int32),
            pltpu.VMEM((sub,), jnp.int32),
            pltpu.VMEM((sub,), jnp.int32),
            pltpu.VMEM((sub,), jnp.int32),
            pltpu.SemaphoreType.DMA,
            pltpu.SemaphoreType.DMA,
            pltpu.SemaphoreType.DMA,
            pltpu.SemaphoreType.DMA,
            pltpu.SemaphoreType.DMA,
        ],
    )
    def dispatch(x_hbm, dst0_hbm, dst1_hbm, einp_hbm,
                 xba, xbb, d0a, d0b, d1a, d1b, sxa, sxb, sca, scb, si):
        wid = lax.axis_index("s") * _NC + lax.axis_index("c")
        xb = [xba, xbb]
        d0 = [d0a, d0b]
        d1 = [d1a, d1b]
        sx = [sxa, sxb]
        sc = [sca, scb]

        def fire_load(s, p):
            base = wid * (sub * nsub) + s * sub
            hx = pltpu.async_copy(x_hbm.at[pl.ds(base, sub), :], xb[p], sx[p])
            h0 = pltpu.async_copy(dst0_hbm.at[wid, s], d0[p], si)
            h1 = pltpu.async_copy(dst1_hbm.at[wid, s], d1[p], si)
            return hx, h0, h1

        pend = {0: fire_load(0, 0)}
        scat_pend = [None, None]
        for s in range(nsub):
            p = s % 2
            if s + 1 < nsub:
                if scat_pend[1 - p] is not None:
                    a, b = scat_pend[1 - p]
                    a.wait()
                    b.wait()
                    scat_pend[1 - p] = None
                pend[s + 1] = fire_load(s + 1, 1 - p)
            hx, h0, h1 = pend.pop(s)
            hx.wait()
            h0.wait()
            h1.wait()
            sc0 = pltpu.async_copy(xb[p], einp_hbm.at[d0[p]], sc[p])
            sc1 = pltpu.async_copy(xb[p], einp_hbm.at[d1[p]], sc[p])
            scat_pend[p] = (sc0, sc1)
        for sp in scat_pend:
            if sp is not None:
                sp[0].wait()
                sp[1].wait()

    return dispatch


def _mlp_body(xe_ref, w1_ref, b1_ref, w2_ref, b2_ref, out_ref, *, nf):
    j = pl.program_id(1)
    x = xe_ref[...]
    hh = jnp.dot(x, w1_ref[0], preferred_element_type=jnp.float32) + b1_ref[0]
    u = 1.5957691216057308 * (hh + 0.044715 * hh * hh * hh)
    hc = hh - hh / (1.0 + jnp.exp(u))
    contrib = jnp.dot(hc, w2_ref[0], preferred_element_type=jnp.float32)

    @pl.when(j == 0)
    def _first():
        out_ref[...] = contrib + b2_ref[0]

    @pl.when(j > 0)
    def _rest():
        out_ref[...] += contrib


def _mlp(einp, w1, b1, w2, b2, cap, bf=1024):
    num_experts, h, f = w1.shape
    nf = f // bf
    return pl.pallas_call(
        functools.partial(_mlp_body, nf=nf),
        grid=(num_experts, nf),
        in_specs=[
            pl.BlockSpec((cap, h), lambda e, j: (e, 0)),
            pl.BlockSpec((1, h, bf), lambda e, j: (e, 0, j)),
            pl.BlockSpec((1, 1, bf), lambda e, j: (e, 0, j)),
            pl.BlockSpec((1, bf, h), lambda e, j: (e, j, 0)),
            pl.BlockSpec((1, 1, h), lambda e, j: (e, 0, 0)),
        ],
        out_specs=pl.BlockSpec((cap, h), lambda e, j: (e, 0)),
        out_shape=jax.ShapeDtypeStruct((num_experts * cap, h), jnp.float32),
    )(einp, w1, b1.reshape(num_experts, 1, f), w2, b2.reshape(num_experts, 1, h))


def _make_combine(n, h, rows_in, sub, nsub):
    mesh = plsc.VectorSubcoreMesh(core_axis_name="c", subcore_axis_name="s")
    nvec = h // _L

    @functools.partial(
        pl.kernel,
        out_type=jax.ShapeDtypeStruct((n, h), jnp.float32),
        mesh=mesh,
        scratch_types=[
            pltpu.VMEM((sub, h), jnp.float32),
            pltpu.VMEM((sub, h), jnp.float32),
            pltpu.VMEM((sub, h), jnp.float32),
            pltpu.VMEM((sub, h), jnp.float32),
            pltpu.VMEM((sub, h), jnp.float32),
            pltpu.VMEM((sub, h), jnp.float32),
            pltpu.VMEM((sub * nsub,), jnp.int32),
            pltpu.VMEM((sub * nsub,), jnp.int32),
            pltpu.VMEM((sub * nsub, 2 * _L), jnp.float32),
            pltpu.SemaphoreType.DMA,
            pltpu.SemaphoreType.DMA,
            pltpu.SemaphoreType.DMA,
            pltpu.SemaphoreType.DMA,
            pltpu.SemaphoreType.DMA,
        ],
    )
    def combine(eo_hbm, src0_hbm, src1_hbm, w_hbm, out_hbm,
                g0a, g0b, g1a, g1b, oba, obb, i0all, i1all, wall,
                sga, sgb, ssa, ssb, si):
        wid = lax.axis_index("s") * _NC + lax.axis_index("c")
        g0 = [g0a, g0b]
        g1 = [g1a, g1b]
        ob = [oba, obb]
        sg = [sga, sgb]
        ss = [ssa, ssb]
        tpw = sub * nsub

        h1 = pltpu.async_copy(src0_hbm.at[wid], i0all, si)
        h2 = pltpu.async_copy(src1_hbm.at[wid], i1all, si)
        h3 = pltpu.async_copy(w_hbm.at[pl.ds(wid * tpw, tpw), :], wall, si)
        h1.wait()
        h2.wait()
        h3.wait()

        def fire_gathers(s, p):
            c0 = pltpu.async_copy(eo_hbm.at[i0all.at[pl.ds(s * sub, sub)]],
                                  g0[p], sg[p])
            c1 = pltpu.async_copy(eo_hbm.at[i1all.at[pl.ds(s * sub, sub)]],
                                  g1[p], sg[p])
            return c0, c1

        def compute(s, p):
            def row_body(r, carry):
                w0vec = wall[s * sub + r, 0:_L]
                w1vec = wall[s * sub + r, _L:2 * _L]
                zero = jnp.zeros((_L,), jnp.float32)

                def vec_body(v, c2):
                    a = g0[p][r, pl.ds(v * _L, _L)]
                    b = g1[p][r, pl.ds(v * _L, _L)]
                    acc = (jnp.where(w0vec != 0.0, w0vec * a, zero)
                           + jnp.where(w1vec != 0.0, w1vec * b, zero))
                    ob[p][r, pl.ds(v * _L, _L)] = acc
                    return c2

                lax.fori_loop(0, nvec, vec_body, 0)
                return carry

            lax.fori_loop(0, sub, row_body, 0)

        pend = {0: fire_gathers(0, 0)}
        store_pend = [None, None]
        for s in range(nsub):
            p = s % 2
            if s + 1 < nsub:
                pend[s + 1] = fire_gathers(s + 1, 1 - p)
            c0, c1 = pend.pop(s)
            c0.wait()
            c1.wait()
            if store_pend[p] is not None:
                store_pend[p].wait()
                store_pend[p] = None
            compute(s, p)
            store_pend[p] = pltpu.async_copy(
                ob[p], out_hbm.at[pl.ds(wid * tpw + s * sub, sub), :], ss[p])
        for sp in store_pend:
            if sp is not None:
                sp.wait()

    return combine


def kernel(x, Wr, W1, b1, W2, b2):
    bsz, t, h = x.shape
    n = bsz * t
    num_experts = Wr.shape[1]
    cap = _capacity_of(n)
    x2d = x.reshape(n, h)
    wr_pad = jnp.zeros((h, _LANES), Wr.dtype).at[:, :num_experts].set(Wr)

    dst0, dst1, src0, src1, wt0, wt1 = _route(x2d, wr_pad, cap, num_experts)

    tpw = n // _NW
    sub_d = 32
    nsub_d = tpw // sub_d
    shp_d = (_NW, nsub_d, sub_d)
    einp = _make_dispatch(n, h, num_experts * cap + 8, sub_d, nsub_d)(
        x2d, dst0.reshape(shp_d), dst1.reshape(shp_d))

    eo = _mlp(einp, W1, b1, W2, b2, cap)

    sub_c = 16
    nsub_c = tpw // sub_c
    shp_c = (_NW, tpw)
    wpack = jnp.concatenate(
        [jnp.broadcast_to(wt0.reshape(n, 1), (n, _L)),
         jnp.broadcast_to(wt1.reshape(n, 1), (n, _L))], axis=1)
    out = _make_combine(n, h, num_experts * cap, sub_c, nsub_c)(
        eo, src0.reshape(shp_c), src1.reshape(shp_c), wpack)
    return out.reshape(bsz, t, h)


# speedup vs baseline: 1.4500x; 1.0514x over previous
import functools
import math

import jax
import jax.numpy as jnp
from jax import lax
from jax.experimental import pallas as pl
from jax.experimental.pallas import tpu as pltpu
from jax.experimental.pallas import tpu_sc as plsc

_CF, _RT = 0.25, 128
_LANES = 128
_NC, _NS, _L = 2, 16, 16
_NW = _NC * _NS


def _capacity_of(num_tokens):
    cap = math.ceil(_CF * num_tokens)
    cap = _RT * math.ceil(cap / _RT)
    return max(1, min(cap, num_tokens))


def _router_body(x_ref, wr_ref, dst0_ref, dst1_ref, src0_ref, src1_ref,
                 w0_ref, w1_ref, carry_ref, *, bm, cap, num_experts):
    @pl.when(pl.program_id(0) == 0)
    def _init():
        carry_ref[...] = jnp.zeros_like(carry_ref)

    x = x_ref[...]
    logits = jnp.dot(x, wr_ref[...], preferred_element_type=jnp.float32)
    lt = jnp.transpose(logits)[:num_experts, :]
    erow = lax.broadcasted_iota(jnp.int32, lt.shape, 0)
    neg = jnp.float32(-jnp.inf)
    m1 = jnp.max(lt, axis=0, keepdims=True)
    i1 = jnp.min(jnp.where(lt == m1, erow, _LANES), axis=0, keepdims=True)
    oh0 = (erow == i1).astype(jnp.float32)
    lm2 = jnp.where(erow == i1, neg, lt)
    m2 = jnp.max(lm2, axis=0, keepdims=True)
    i2 = jnp.min(jnp.where(lm2 == m2, erow, _LANES), axis=0, keepdims=True)
    oh1 = (erow == i2).astype(jnp.float32)
    ed = jnp.exp(m2 - m1)
    w0 = 1.0 / (1.0 + ed)
    w1 = ed * w0
    oh = oh0 + oh1
    r = lax.broadcasted_iota(jnp.int32, (bm, bm), 0)
    c = lax.broadcasted_iota(jnp.int32, (bm, bm), 1)
    triu = (r < c).astype(jnp.float32)
    excl = (jnp.dot(oh, triu, preferred_element_type=jnp.float32)
            + carry_ref[:num_experts, 0:1])
    rank0 = jnp.sum(excl * oh0, axis=0).astype(jnp.int32)
    rank1 = jnp.sum(excl * oh1, axis=0).astype(jnp.int32)
    carry_ref[:num_experts, 0:1] += jnp.sum(oh, axis=1, keepdims=True)

    e0 = i1[0]
    e1 = i2[0]
    k0 = rank0 < cap
    k1 = rank1 < cap
    f0 = e0 * cap + rank0
    f1 = e1 * cap + rank1
    w0 = w0[0]
    w1 = w1[0]
    trash = jnp.int32(num_experts * cap)
    dst0_ref[0, 0, :] = jnp.where(k0, f0, trash)
    dst1_ref[0, 0, :] = jnp.where(k1, f1, trash)
    src0_ref[0, 0, :] = jnp.where(k0, f0, 0)
    src1_ref[0, 0, :] = jnp.where(k1, f1, 0)
    w0_ref[0, 0, :] = jnp.where(k0, w0, 0.0)
    w1_ref[0, 0, :] = jnp.where(k1, w1, 0.0)


def _route(x2d, wr_pad, cap, num_experts, bm=512):
    n, h = x2d.shape
    nb = n // bm
    ispec = jnp.int32
    out_shapes = [jax.ShapeDtypeStruct((nb, 1, bm), ispec) for _ in range(4)]
    out_shapes += [jax.ShapeDtypeStruct((nb, 1, bm), jnp.float32) for _ in range(2)]
    small = pl.BlockSpec((1, 1, bm), lambda i: (i, 0, 0))
    return pl.pallas_call(
        functools.partial(_router_body, bm=bm, cap=cap, num_experts=num_experts),
        grid=(nb,),
        in_specs=[
            pl.BlockSpec((bm, h), lambda i: (i, 0)),
            pl.BlockSpec((h, _LANES), lambda i: (0, 0)),
        ],
        out_specs=[small] * 6,
        out_shape=out_shapes,
        scratch_shapes=[pltpu.VMEM((num_experts, _LANES), jnp.float32)],
    )(x2d, wr_pad)


def _make_dispatch(n, h, rows_out, sub, nsub):
    mesh = plsc.VectorSubcoreMesh(core_axis_name="c", subcore_axis_name="s")

    @functools.partial(
        pl.kernel,
        out_type=jax.ShapeDtypeStruct((rows_out, h), jnp.float32),
        mesh=mesh,
        scratch_types=[
            pltpu.VMEM((sub, h), jnp.float32),
            pltpu.VMEM((sub, h), jnp.float32),
            pltpu.VMEM((sub,), jnp.int32),
            pltpu.VMEM((sub,), jnp.int32),
            pltpu.VMEM((sub,), jnp.int32),
            pltpu.VMEM((sub,), jnp.int32),
            pltpu.SemaphoreType.DMA,
            pltpu.SemaphoreType.DMA,
            pltpu.SemaphoreType.DMA,
            pltpu.SemaphoreType.DMA,
            pltpu.SemaphoreType.DMA,
        ],
    )
    def dispatch(x_hbm, dst0_hbm, dst1_hbm, einp_hbm,
                 xba, xbb, d0a, d0b, d1a, d1b, sxa, sxb, sca, scb, si):
        wid = lax.axis_index("s") * _NC + lax.axis_index("c")
        xb = [xba, xbb]
        d0 = [d0a, d0b]
        d1 = [d1a, d1b]
        sx = [sxa, sxb]
        sc = [sca, scb]

        def fire_load(s, p):
            base = wid * (sub * nsub) + s * sub
            hx = pltpu.async_copy(x_hbm.at[pl.ds(base, sub), :], xb[p], sx[p])
            h0 = pltpu.async_copy(dst0_hbm.at[wid, s], d0[p], si)
            h1 = pltpu.async_copy(dst1_hbm.at[wid, s], d1[p], si)
            return hx, h0, h1

        pend = {0: fire_load(0, 0)}
        scat_pend = [None, None]
        for s in range(nsub):
            p = s % 2
            if s + 1 < nsub:
                if scat_pend[1 - p] is not None:
                    a, b = scat_pend[1 - p]
                    a.wait()
                    b.wait()
                    scat_pend[1 - p] = None
                pend[s + 1] = fire_load(s + 1, 1 - p)
            hx, h0, h1 = pend.pop(s)
            hx.wait()
            h0.wait()
            h1.wait()
            sc0 = pltpu.async_copy(xb[p], einp_hbm.at[d0[p]], sc[p])
            sc1 = pltpu.async_copy(xb[p], einp_hbm.at[d1[p]], sc[p])
            scat_pend[p] = (sc0, sc1)
        for sp in scat_pend:
            if sp is not None:
                sp[0].wait()
                sp[1].wait()

    return dispatch


def _mlp_body(xe_ref, w1_ref, b1_ref, w2_ref, b2_ref, out_ref, *, nf):
    j = pl.program_id(1)
    x = xe_ref[...]
    hh = jnp.dot(x, w1_ref[0], preferred_element_type=jnp.float32) + b1_ref[0]
    u = 1.5957691216057308 * (hh + 0.044715 * hh * hh * hh)
    hc = hh - hh / (1.0 + jnp.exp(u))
    contrib = jnp.dot(hc, w2_ref[0], preferred_element_type=jnp.float32)

    @pl.when(j == 0)
    def _first():
        out_ref[...] = contrib + b2_ref[0]

    @pl.when(j > 0)
    def _rest():
        out_ref[...] += contrib


def _mlp(einp, w1, b1, w2, b2, cap, bf=1024):
    num_experts, h, f = w1.shape
    nf = f // bf
    return pl.pallas_call(
        functools.partial(_mlp_body, nf=nf),
        grid=(num_experts, nf),
        in_specs=[
            pl.BlockSpec((cap, h), lambda e, j: (e, 0)),
            pl.BlockSpec((1, h, bf), lambda e, j: (e, 0, j)),
            pl.BlockSpec((1, 1, bf), lambda e, j: (e, 0, j)),
            pl.BlockSpec((1, bf, h), lambda e, j: (e, j, 0)),
            pl.BlockSpec((1, 1, h), lambda e, j: (e, 0, 0)),
        ],
        out_specs=pl.BlockSpec((cap, h), lambda e, j: (e, 0)),
        out_shape=jax.ShapeDtypeStruct((num_experts * cap, h), jnp.float32),
    )(einp, w1, b1.reshape(num_experts, 1, f), w2, b2.reshape(num_experts, 1, h))


def _make_combine(n, h, rows_in, sub, nsub):
    mesh = plsc.VectorSubcoreMesh(core_axis_name="c", subcore_axis_name="s")
    nvec = h // _L

    @functools.partial(
        pl.kernel,
        out_type=jax.ShapeDtypeStruct((n, h), jnp.float32),
        mesh=mesh,
        scratch_types=[
            pltpu.VMEM((sub, h), jnp.float32),
            pltpu.VMEM((sub, h), jnp.float32),
            pltpu.VMEM((sub, h), jnp.float32),
            pltpu.VMEM((sub, h), jnp.float32),
            pltpu.VMEM((sub, h), jnp.float32),
            pltpu.VMEM((sub, h), jnp.float32),
            pltpu.VMEM((sub * nsub,), jnp.int32),
            pltpu.VMEM((sub * nsub,), jnp.int32),
            pltpu.VMEM((sub * nsub, 2 * _L), jnp.float32),
            pltpu.SemaphoreType.DMA,
            pltpu.SemaphoreType.DMA,
            pltpu.SemaphoreType.DMA,
            pltpu.SemaphoreType.DMA,
            pltpu.SemaphoreType.DMA,
        ],
    )
    def combine(eo_hbm, src0_hbm, src1_hbm, w_hbm, out_hbm,
                g0a, g0b, g1a, g1b, oba, obb, i0all, i1all, wall,
                sga, sgb, ssa, ssb, si):
        wid = lax.axis_index("s") * _NC + lax.axis_index("c")
        g0 = [g0a, g0b]
        g1 = [g1a, g1b]
        ob = [oba, obb]
        sg = [sga, sgb]
        ss = [ssa, ssb]
        tpw = sub * nsub

        h1 = pltpu.async_copy(src0_hbm.at[wid], i0all, si)
        h2 = pltpu.async_copy(src1_hbm.at[wid], i1all, si)
        h3 = pltpu.async_copy(w_hbm.at[pl.ds(wid * tpw, tpw), :], wall, si)
        h1.wait()
        h2.wait()
        h3.wait()

        def fire_gathers(s, p):
            c0 = pltpu.async_copy(eo_hbm.at[i0all.at[pl.ds(s * sub, sub)]],
                                  g0[p], sg[p])
            c1 = pltpu.async_copy(eo_hbm.at[i1all.at[pl.ds(s * sub, sub)]],
                                  g1[p], sg[p])
            return c0, c1

        def compute(s, p):
            def row_body(r, carry):
                w0vec = wall[s * sub + r, 0:_L]
                w1vec = wall[s * sub + r, _L:2 * _L]
                zero = jnp.zeros((_L,), jnp.float32)

                def vec_body(v, c2):
                    a = g0[p][r, pl.ds(v * _L, _L)]
                    b = g1[p][r, pl.ds(v * _L, _L)]
                    acc = (jnp.where(w0vec != 0.0, w0vec * a, zero)
                           + jnp.where(w1vec != 0.0, w1vec * b, zero))
                    ob[p][r, pl.ds(v * _L, _L)] = acc
                    return c2

                lax.fori_loop(0, nvec, vec_body, 0)
                return carry

            lax.fori_loop(0, sub, row_body, 0)

        pend = {0: fire_gathers(0, 0)}
        store_pend = [None, None]
        for s in range(nsub):
            p = s % 2
            if s + 1 < nsub:
                pend[s + 1] = fire_gathers(s + 1, 1 - p)
            c0, c1 = pend.pop(s)
            c0.wait()
            c1.wait()
            if store_pend[p] is not None:
                store_pend[p].wait()
                store_pend[p] = None
            compute(s, p)
            store_pend[p] = pltpu.async_copy(
                ob[p], out_hbm.at[pl.ds(wid * tpw + s * sub, sub), :], ss[p])
        for sp in store_pend:
            if sp is not None:
                sp.wait()

    return combine


def kernel(x, Wr, W1, b1, W2, b2):
    bsz, t, h = x.shape
    n = bsz * t
    num_experts = Wr.shape[1]
    cap = _capacity_of(n)
    x2d = x.reshape(n, h)
    wr_pad = jnp.zeros((h, _LANES), Wr.dtype).at[:, :num_experts].set(Wr)

    dst0, dst1, src0, src1, wt0, wt1 = _route(x2d, wr_pad, cap, num_experts)

    tpw = n // _NW
    sub_d = 32
    nsub_d = tpw // sub_d
    shp_d = (_NW, nsub_d, sub_d)
    einp = _make_dispatch(n, h, num_experts * cap + 8, sub_d, nsub_d)(
        x2d, dst0.reshape(shp_d), dst1.reshape(shp_d))

    eo = _mlp(einp, W1, b1, W2, b2, cap)

    sub_c = 16
    nsub_c = tpw // sub_c
    shp_c = (_NW, tpw)
    wpack = jnp.concatenate(
        [jnp.broadcast_to(wt0.reshape(n, 1), (n, _L)),
         jnp.broadcast_to(wt1.reshape(n, 1), (n, _L))], axis=1)
    out = _make_combine(n, h, num_experts * cap, sub_c, nsub_c)(
        eo, src0.reshape(shp_c), src1.reshape(shp_c), wpack)
    return out.reshape(bsz, t, h)
